# speedup vs baseline: 1.6413x; 1.6413x over previous
import functools

import jax
import jax.numpy as jnp
from jax import lax
from jax.experimental import pallas as pl
from jax.experimental.pallas import tpu as pltpu
from jax.experimental.pallas import tpu_sc as plsc

NC = 2
NS = 16
NW = NC * NS


@functools.lru_cache(maxsize=None)
def _make_gather(n: int, vocab: int, dim: int):
    rows_per_w = n // NW
    chunk = 32
    nchunk = rows_per_w // chunk
    mesh = plsc.VectorSubcoreMesh(core_axis_name="c", subcore_axis_name="s")

    @functools.partial(
        pl.kernel,
        mesh=mesh,
        out_type=jax.ShapeDtypeStruct((n, dim), jnp.float32),
        scratch_types=[
            pltpu.VMEM((rows_per_w,), jnp.int32),
            pltpu.VMEM((2, chunk, dim), jnp.float32),
            pltpu.SemaphoreType.DMA,
            pltpu.SemaphoreType.DMA,
            pltpu.SemaphoreType.DMA,
            pltpu.SemaphoreType.DMA,
        ],
    )
    def gather(ids_hbm, table_hbm, out_hbm, idx_v, rows_v, g0, g1, s0, s1):
        wid = lax.axis_index("s") * NC + lax.axis_index("c")
        base = wid * rows_per_w
        pltpu.sync_copy(ids_hbm.at[pl.ds(base, rows_per_w)], idx_v)
        gsem = (g0, g1)
        ssem = (s0, s1)

        pending_gather = [None, None]
        pending_store = [None, None]
        pending_gather[0] = pltpu.async_copy(
            table_hbm.at[idx_v.at[pl.ds(0, chunk)]], rows_v.at[0], gsem[0])
        for c in range(nchunk):
            buf = c % 2
            nbuf = (c + 1) % 2
            if c + 1 < nchunk:
                if pending_store[nbuf] is not None:
                    pending_store[nbuf].wait()
                    pending_store[nbuf] = None
                pending_gather[nbuf] = pltpu.async_copy(
                    table_hbm.at[idx_v.at[pl.ds((c + 1) * chunk, chunk)]],
                    rows_v.at[nbuf], gsem[nbuf])
            pending_gather[buf].wait()
            pending_store[buf] = pltpu.async_copy(
                rows_v.at[buf], out_hbm.at[pl.ds(base + c * chunk, chunk)],
                ssem[buf])
        for st in pending_store:
            if st is not None:
                st.wait()

    return gather


def kernel(input_ids, embed_table):
    b, s = input_ids.shape
    vocab, dim = embed_table.shape
    n = b * s
    flat = input_ids.reshape(n).astype(jnp.int32)
    out = _make_gather(n, vocab, dim)(flat, embed_table)
    return out.reshape(b, s, dim)


# speedup vs baseline: 1.6500x; 1.0053x over previous
import functools

import jax
import jax.numpy as jnp
from jax import lax
from jax.experimental import pallas as pl
from jax.experimental.pallas import tpu as pltpu
from jax.experimental.pallas import tpu_sc as plsc

NC = 2
NS = 16
NW = NC * NS


@functools.lru_cache(maxsize=None)
def _make_gather(n: int, vocab: int, dim: int):
    rows_per_w = n // NW
    chunk = 32
    nbuf = 3
    nchunk = rows_per_w // chunk
    mesh = plsc.VectorSubcoreMesh(core_axis_name="c", subcore_axis_name="s")

    @functools.partial(
        pl.kernel,
        mesh=mesh,
        out_type=jax.ShapeDtypeStruct((n, dim), jnp.float32),
        scratch_types=[
            pltpu.VMEM((rows_per_w,), jnp.int32),
            pltpu.VMEM((nbuf, chunk, dim), jnp.float32),
        ]
        + [pltpu.SemaphoreType.DMA] * (2 * nbuf),
    )
    def gather(ids_hbm, table_hbm, out_hbm, idx_v, rows_v, *sems):
        gsem = sems[:nbuf]
        ssem = sems[nbuf:]
        wid = lax.axis_index("s") * NC + lax.axis_index("c")
        base = wid * rows_per_w
        pltpu.sync_copy(ids_hbm.at[pl.ds(base, rows_per_w)], idx_v)

        def start_gather(c):
            return pltpu.async_copy(
                table_hbm.at[idx_v.at[pl.ds(c * chunk, chunk)]],
                rows_v.at[c % nbuf], gsem[c % nbuf])

        pending_gather = [start_gather(b) for b in range(min(nbuf, nchunk))]
        pending_gather += [None] * (nbuf - len(pending_gather))
        pending_store = [None] * nbuf
        for c in range(nchunk):
            g = c - 1 + nbuf
            if c >= 1 and g < nchunk:
                pb = (c - 1) % nbuf
                pending_store[pb].wait()
                pending_store[pb] = None
                pending_gather[pb] = start_gather(g)
            buf = c % nbuf
            pending_gather[buf].wait()
            pending_store[buf] = pltpu.async_copy(
                rows_v.at[buf], out_hbm.at[pl.ds(base + c * chunk, chunk)],
                ssem[buf])
        for st in pending_store:
            if st is not None:
                st.wait()

    return gather


def kernel(input_ids, embed_table):
    b, s = input_ids.shape
    vocab, dim = embed_table.shape
    n = b * s
    flat = input_ids.reshape(n).astype(jnp.int32)
    out = _make_gather(n, vocab, dim)(flat, embed_table)
    return out.reshape(b, s, dim)
